# baseline (speedup 1.0000x reference)
import functools

import jax
import jax.numpy as jnp
from jax import lax
from jax.experimental import pallas as pl
from jax.experimental.pallas import tpu as pltpu
from jax.experimental.pallas import tpu_sc as plsc

_EMB = 32
_BATCH = 16384
_HIST = 200
_B = _BATCH * _HIST
_NW = 32
_BPW = _B // _NW
_C = 800
_G = _BPW // _C
_NBUF = 4
_K = 2
_RPC = _C // _HIST

_mesh = plsc.VectorSubcoreMesh(core_axis_name="c", subcore_axis_name="s")


@functools.partial(
    pl.kernel,
    mesh=_mesh,
    out_type=jax.ShapeDtypeStruct((_BATCH, _HIST, _EMB), jnp.float32),
    scratch_types=[
        pltpu.VMEM((_NBUF * _C,), jnp.int32),
        pltpu.VMEM((_NBUF, _C, _EMB), jnp.float32),
    ] + [pltpu.SemaphoreType.DMA] * (3 * _NBUF),
    compiler_params=pltpu.CompilerParams(use_tc_tiling_on_sc=False),
)
def _emb_lookup(idx_hbm, table_hbm, out_hbm, idx_v, rows_v, *sems):
    sem_g = sems[:_NBUF]
    sem_w = sems[_NBUF:2 * _NBUF]
    sem_i = sems[2 * _NBUF:]
    wid = lax.axis_index("s") * 2 + lax.axis_index("c")
    base = wid * _BPW

    def prefetch_idx(j, b):
        pltpu.async_copy(
            idx_hbm.at[pl.ds(base + j * _C, _C)],
            idx_v.at[pl.ds(b * _C, _C)], sem_i[b],
        )

    def fire_gather(j, b):
        pltpu.make_async_copy(
            idx_hbm.at[pl.ds(base + j * _C, _C)],
            idx_v.at[pl.ds(b * _C, _C)], sem_i[b],
        ).wait()
        pltpu.async_copy(table_hbm.at[idx_v.at[pl.ds(b * _C, _C)]], rows_v.at[b], sem_g[b])

    for j in range(_K + 1):
        prefetch_idx(j, j)
    for j in range(_K):
        fire_gather(j, j)

    def group(gg, carry):
        for phase in range(_NBUF):
            i = gg * _NBUF + phase
            bi = phase
            bj = (phase + _K) % _NBUF
            bp = (phase + _K + 1) % _NBUF
            j = i + _K

            @pl.when(i + _K + 1 < _G)
            def _():
                prefetch_idx(i + _K + 1, bp)

            @pl.when(j < _G)
            def _():
                @pl.when(j >= _NBUF)
                def _():
                    for r in range(_RPC):
                        pltpu.make_async_copy(
                            rows_v.at[bj, pl.ds(r * _HIST, _HIST)],
                            out_hbm.at[0], sem_w[bj],
                        ).wait()

                fire_gather(j, bj)

            pltpu.make_async_copy(
                table_hbm.at[idx_v.at[pl.ds(bi * _C, _C)]], rows_v.at[bi], sem_g[bi]
            ).wait()
            gb = wid * (_BPW // _HIST) + i * _RPC
            for r in range(_RPC):
                pltpu.async_copy(
                    rows_v.at[bi, pl.ds(r * _HIST, _HIST)], out_hbm.at[gb + r],
                    sem_w[bi],
                )
        return carry

    lax.fori_loop(0, _G // _NBUF, group, 0)

    for b in range(_NBUF):
        for r in range(_RPC):
            pltpu.make_async_copy(
                rows_v.at[b, pl.ds(r * _HIST, _HIST)], out_hbm.at[0], sem_w[b]
            ).wait()


def kernel(input, weight):
    idx = input.reshape(_B).astype(jnp.int32)
    return _emb_lookup(idx, weight)


# speedup vs baseline: 1.6787x; 1.6787x over previous
import functools

import jax
import jax.numpy as jnp
from jax import lax
from jax.experimental import pallas as pl
from jax.experimental.pallas import tpu as pltpu
from jax.experimental.pallas import tpu_sc as plsc

_EMB = 32
_BATCH = 16384
_HIST = 200
_B = _BATCH * _HIST
_NW = 32
_BPW = _B // _NW
_C = 800
_G = _BPW // _C
_NBUF = 4
_K = 2
_PAD = 128

_mesh = plsc.VectorSubcoreMesh(core_axis_name="c", subcore_axis_name="s")


@functools.partial(
    pl.kernel,
    mesh=_mesh,
    out_type=jax.ShapeDtypeStruct((_B, _PAD), jnp.float32),
    scratch_types=[
        pltpu.VMEM((_NBUF * _C,), jnp.int32),
        pltpu.VMEM((_NBUF, _C, _EMB), jnp.float32),
    ] + [pltpu.SemaphoreType.DMA] * (3 * _NBUF),
    compiler_params=pltpu.CompilerParams(use_tc_tiling_on_sc=False),
)
def _gather_padded(idx_hbm, table_hbm, out_hbm, idx_v, rows_v, *sems):
    sem_g = sems[:_NBUF]
    sem_w = sems[_NBUF:2 * _NBUF]
    sem_i = sems[2 * _NBUF:]
    wid = lax.axis_index("s") * 2 + lax.axis_index("c")
    base = wid * _BPW

    def prefetch_idx(j, b):
        pltpu.async_copy(
            idx_hbm.at[pl.ds(base + j * _C, _C)],
            idx_v.at[pl.ds(b * _C, _C)], sem_i[b],
        )

    def fire_gather(j, b):
        pltpu.make_async_copy(
            idx_hbm.at[pl.ds(base + j * _C, _C)],
            idx_v.at[pl.ds(b * _C, _C)], sem_i[b],
        ).wait()
        pltpu.async_copy(table_hbm.at[idx_v.at[pl.ds(b * _C, _C)]], rows_v.at[b], sem_g[b])

    def wb_dst(i):
        return out_hbm.at[pl.ds(base + i * _C, _C), pl.ds(0, _EMB)]

    for j in range(_K + 1):
        prefetch_idx(j, j)
    for j in range(_K):
        fire_gather(j, j)

    def group(gg, carry):
        for phase in range(_NBUF):
            i = gg * _NBUF + phase
            bi = phase
            bj = (phase + _K) % _NBUF
            bp = (phase + _K + 1) % _NBUF
            j = i + _K

            @pl.when(i + _K + 1 < _G)
            def _():
                prefetch_idx(i + _K + 1, bp)

            @pl.when(j < _G)
            def _():
                @pl.when(j >= _NBUF)
                def _():
                    pltpu.make_async_copy(
                        rows_v.at[bj], wb_dst(0), sem_w[bj],
                    ).wait()

                fire_gather(j, bj)

            pltpu.make_async_copy(
                table_hbm.at[idx_v.at[pl.ds(bi * _C, _C)]], rows_v.at[bi], sem_g[bi]
            ).wait()
            pltpu.async_copy(rows_v.at[bi], wb_dst(i), sem_w[bi])
        return carry

    lax.fori_loop(0, _G // _NBUF, group, 0)

    for b in range(_NBUF):
        pltpu.make_async_copy(rows_v.at[b], wb_dst(0), sem_w[b]).wait()


def _transpose_body(x_ref, o_ref):
    xt = jnp.transpose(x_ref[0], (1, 0))
    o_ref[0] = xt[0:_EMB, :]


def _transpose_slabs(y):
    return pl.pallas_call(
        _transpose_body,
        grid=(_HIST,),
        in_specs=[pl.BlockSpec((1, _BATCH, _PAD), lambda i: (i, 0, 0))],
        out_specs=pl.BlockSpec((1, _EMB, _BATCH), lambda i: (i, 0, 0)),
        out_shape=jax.ShapeDtypeStruct((_HIST, _EMB, _BATCH), jnp.float32),
    )(y)


def kernel(input, weight):
    idx = jnp.transpose(input).reshape(_B).astype(jnp.int32)
    yp = _gather_padded(idx, weight)
    out3 = _transpose_slabs(yp.reshape(_HIST, _BATCH, _PAD))
    return out3.transpose(2, 0, 1)
